# Initial kernel scaffold; baseline (speedup 1.0000x reference)
import jax
import jax.numpy as jnp
from jax.experimental import pallas as pl


def kernel(card_indices, stages, rank_emb, suit_emb, stage_emb):
    raise NotImplementedError("write your pallas kernel here")


# speedup vs baseline: 4.9416x; 4.9416x over previous
import functools

import jax
import jax.numpy as jnp
from jax import lax
from jax.experimental import pallas as pl
from jax.experimental.pallas import tpu as pltpu
from jax.experimental.pallas import tpu_sc as plsc

D_MODEL = 256
T_ROWS = 256
NUM_CORES = 2
NUM_SUBCORES = 16
NUM_WORKERS = NUM_CORES * NUM_SUBCORES
CHUNK = 128


def _build_table_kernel(rank_ref, suit_ref, stage_ref, t_ref):
    rows = lax.broadcasted_iota(jnp.int32, (T_ROWS, 1), 0)
    card = rows // 4
    stg = rows % 4
    rank = card % 13
    suit = card // 13
    valid = card < 52
    acc = jnp.zeros((T_ROWS, D_MODEL), jnp.float32)
    for k in range(13):
        acc += jnp.where(rank == k, 1.0, 0.0) * rank_ref[k, :][None, :]
    for k in range(4):
        acc += jnp.where(suit == k, 1.0, 0.0) * suit_ref[k, :][None, :]
        acc += jnp.where(stg == k, 1.0, 0.0) * stage_ref[k, :][None, :]
    t_ref[...] = jnp.where(valid, acc, 0.0)


def _build_table(rank_emb, suit_emb, stage_emb):
    return pl.pallas_call(
        _build_table_kernel,
        out_shape=jax.ShapeDtypeStruct((T_ROWS, D_MODEL), jnp.float32),
    )(rank_emb, suit_emb, stage_emb)


def _make_sc_gather(n_pos):
    assert n_pos % (NUM_WORKERS * CHUNK) == 0
    per_worker = n_pos // NUM_WORKERS
    n_chunks = per_worker // CHUNK
    mesh = plsc.VectorSubcoreMesh(core_axis_name="c", subcore_axis_name="s")

    @functools.partial(
        pl.kernel,
        out_type=jax.ShapeDtypeStruct((n_pos, D_MODEL), jnp.float32),
        mesh=mesh,
        scratch_types=[
            pltpu.VMEM((CHUNK,), jnp.int32),
            pltpu.VMEM((CHUNK,), jnp.int32),
            pltpu.VMEM((CHUNK,), jnp.int32),
            pltpu.VMEM((CHUNK, D_MODEL), jnp.float32),
            pltpu.SemaphoreType.DMA,
        ],
    )
    def sc_gather(cards_hbm, stg_hbm, t_hbm, out_hbm,
                  card_v, stg_v, idx_v, rows_v, gsem):
        wid = lax.axis_index("s") * NUM_CORES + lax.axis_index("c")
        base = wid * per_worker

        def step(g, carry):
            off = base + g * CHUNK
            pltpu.sync_copy(cards_hbm.at[pl.ds(off, CHUNK)], card_v)
            pltpu.sync_copy(stg_hbm.at[pl.ds(off, CHUNK)], stg_v)
            for j in range(CHUNK // 16):
                sl = pl.ds(j * 16, 16)
                c = card_v[sl]
                s = stg_v[sl]
                valid = (c >= 0) & (c < 52)
                cc = jnp.where(valid, c, 52)
                ss = jnp.clip(s, 0, 3)
                idx_v[sl] = cc * 4 + ss
            pltpu.async_copy(t_hbm.at[idx_v], rows_v, gsem).wait()
            pltpu.sync_copy(rows_v, out_hbm.at[pl.ds(off, CHUNK)])
            return carry

        lax.fori_loop(0, n_chunks, step, 0)

    return sc_gather


def kernel(card_indices, stages, rank_emb, suit_emb, stage_emb):
    batch, seq = card_indices.shape
    n_pos = batch * seq
    cards = card_indices.reshape(n_pos).astype(jnp.int32)
    stg = stages.reshape(n_pos).astype(jnp.int32)
    table = _build_table(rank_emb, suit_emb, stage_emb)
    out = _make_sc_gather(n_pos)(cards, stg, table)
    return out.reshape(batch, seq, D_MODEL)


# speedup vs baseline: 5.0266x; 1.0172x over previous
import functools

import jax
import jax.numpy as jnp
from jax import lax
from jax.experimental import pallas as pl
from jax.experimental.pallas import tpu as pltpu
from jax.experimental.pallas import tpu_sc as plsc

D_MODEL = 256
T_ROWS = 256
NUM_CORES = 2
NUM_SUBCORES = 16
NUM_WORKERS = NUM_CORES * NUM_SUBCORES
CHUNK = 128


def _build_table_kernel(rank_ref, suit_ref, stage_ref, t_ref):
    rows = lax.broadcasted_iota(jnp.int32, (T_ROWS, 1), 0)
    card = rows // 4
    stg = rows % 4
    rank = card % 13
    suit = card // 13
    valid = card < 52
    acc = jnp.zeros((T_ROWS, D_MODEL), jnp.float32)
    for k in range(13):
        acc += jnp.where(rank == k, 1.0, 0.0) * rank_ref[k, :][None, :]
    for k in range(4):
        acc += jnp.where(suit == k, 1.0, 0.0) * suit_ref[k, :][None, :]
        acc += jnp.where(stg == k, 1.0, 0.0) * stage_ref[k, :][None, :]
    t_ref[...] = jnp.where(valid, acc, 0.0)


def _build_table(rank_emb, suit_emb, stage_emb):
    return pl.pallas_call(
        _build_table_kernel,
        out_shape=jax.ShapeDtypeStruct((T_ROWS, D_MODEL), jnp.float32),
    )(rank_emb, suit_emb, stage_emb)


NBUF = 2


def _make_sc_gather(n_pos):
    assert n_pos % (NUM_WORKERS * CHUNK * NBUF) == 0
    per_worker = n_pos // NUM_WORKERS
    n_chunks = per_worker // CHUNK
    mesh = plsc.VectorSubcoreMesh(core_axis_name="c", subcore_axis_name="s")

    scratch = []
    for _ in range(NBUF):
        scratch += [
            pltpu.VMEM((CHUNK,), jnp.int32),
            pltpu.VMEM((CHUNK,), jnp.int32),
            pltpu.VMEM((CHUNK,), jnp.int32),
            pltpu.VMEM((CHUNK, D_MODEL), jnp.float32),
            pltpu.SemaphoreType.DMA,
            pltpu.SemaphoreType.DMA,
            pltpu.SemaphoreType.DMA,
        ]

    @functools.partial(
        pl.kernel,
        out_type=jax.ShapeDtypeStruct((n_pos, D_MODEL), jnp.float32),
        mesh=mesh,
        scratch_types=scratch,
    )
    def sc_gather(cards_hbm, stg_hbm, t_hbm, out_hbm, *bufs):
        card_v = [bufs[7 * b + 0] for b in range(NBUF)]
        stg_v = [bufs[7 * b + 1] for b in range(NBUF)]
        idx_v = [bufs[7 * b + 2] for b in range(NBUF)]
        rows_v = [bufs[7 * b + 3] for b in range(NBUF)]
        isem = [bufs[7 * b + 4] for b in range(NBUF)]
        gsem = [bufs[7 * b + 5] for b in range(NBUF)]
        osem = [bufs[7 * b + 6] for b in range(NBUF)]
        wid = lax.axis_index("s") * NUM_CORES + lax.axis_index("c")
        base = wid * per_worker

        def fire_in(off, b):
            pltpu.async_copy(cards_hbm.at[pl.ds(off, CHUNK)], card_v[b], isem[b])
            pltpu.async_copy(stg_hbm.at[pl.ds(off, CHUNK)], stg_v[b], isem[b])

        for b in range(NBUF):
            fire_in(base + b * CHUNK, b)

        def group(g, carry):
            for b in range(NBUF):
                c = g * NBUF + b
                off = base + c * CHUNK
                pltpu.make_async_copy(
                    cards_hbm.at[pl.ds(off, CHUNK)], card_v[b], isem[b]).wait()
                pltpu.make_async_copy(
                    stg_hbm.at[pl.ds(off, CHUNK)], stg_v[b], isem[b]).wait()
                for j in range(CHUNK // 16):
                    sl = pl.ds(j * 16, 16)
                    cv = card_v[b][sl]
                    sv = stg_v[b][sl]
                    valid = (cv >= 0) & (cv < 52)
                    cc = jnp.where(valid, cv, 52)
                    ss = jnp.clip(sv, 0, 3)
                    idx_v[b][sl] = cc * 4 + ss

                @pl.when(g > 0)
                def _wait_prev_scatter():
                    pltpu.make_async_copy(
                        rows_v[b], out_hbm.at[pl.ds(base, CHUNK)], osem[b]).wait()

                gath = pltpu.async_copy(t_hbm.at[idx_v[b]], rows_v[b], gsem[b])

                @pl.when(c + NBUF < n_chunks)
                def _fire_next_in():
                    fire_in(off + NBUF * CHUNK, b)

                gath.wait()
                pltpu.async_copy(rows_v[b], out_hbm.at[pl.ds(off, CHUNK)], osem[b])
            return carry

        lax.fori_loop(0, n_chunks // NBUF, group, 0)
        for b in range(NBUF):
            pltpu.make_async_copy(
                rows_v[b], out_hbm.at[pl.ds(base, CHUNK)], osem[b]).wait()

    return sc_gather


def kernel(card_indices, stages, rank_emb, suit_emb, stage_emb):
    batch, seq = card_indices.shape
    n_pos = batch * seq
    cards = card_indices.reshape(n_pos).astype(jnp.int32)
    stg = stages.reshape(n_pos).astype(jnp.int32)
    table = _build_table(rank_emb, suit_emb, stage_emb)
    out = _make_sc_gather(n_pos)(cards, stg, table)
    return out.reshape(batch, seq, D_MODEL)
